# Initial kernel scaffold; baseline (speedup 1.0000x reference)
import jax
import jax.numpy as jnp
from jax.experimental import pallas as pl


def kernel(embedding, seq_len):
    raise NotImplementedError("write your pallas kernel here")


# speedup vs baseline: 1.4393x; 1.4393x over previous
import functools

import jax
import jax.numpy as jnp
from jax import lax
from jax.experimental import pallas as pl
from jax.experimental.pallas import tpu as pltpu
from jax.experimental.pallas import tpu_sc as plsc

S = 512
D = 768
NW = 32
RPW = S // NW
JB = 64
NJB = S // JB
CHUNK = JB + RPW


def kernel(embedding, seq_len):
    del seq_len

    emb_padded = jnp.concatenate(
        [embedding, jnp.zeros((1, D), embedding.dtype)], axis=0
    )

    mesh = plsc.VectorSubcoreMesh(core_axis_name="c", subcore_axis_name="s")

    @functools.partial(
        pl.kernel,
        mesh=mesh,
        out_type=jax.ShapeDtypeStruct((S, S, D), jnp.float32),
        scratch_types=[
            pltpu.VMEM((CHUNK, D), jnp.float32),
            pltpu.SemaphoreType.DMA,
        ],
        compiler_params=pltpu.CompilerParams(use_tc_tiling_on_sc=False),
    )
    def sliding_copy(emb_hbm, out_hbm, chunk_v, sem):
        wid = lax.axis_index("s") * 2 + lax.axis_index("c")
        i0 = wid * RPW

        def jb_body(jb, carry):
            j0 = jb * JB
            base = j0 - i0 + (S - RPW)
            pltpu.async_copy(
                emb_hbm.at[pl.ds(base, CHUNK), :], chunk_v, sem
            ).wait()
            for r in range(RPW):
                pltpu.sync_copy(
                    chunk_v.at[pl.ds(RPW - 1 - r, JB), :],
                    out_hbm.at[i0 + r, pl.ds(j0, JB), :],
                )
            return carry

        lax.fori_loop(0, NJB, jb_body, 0)

    return sliding_copy(emb_padded)
